# Initial kernel scaffold; baseline (speedup 1.0000x reference)
import jax
import jax.numpy as jnp
from jax.experimental import pallas as pl


def kernel(x, edge_index, edge_attr, batch, c1w, c1b, c2w, c2b, bn_g, bn_b, g1_wl, g1_bl, g1_wr, g1_br, g1_we, g1_att, g1_bias, g2_wl, g2_bl, g2_wr, g2_br, g2_we, g2_att, g2_bias, fc1_w, fc1_b, fc2_w, fc2_b):
    raise NotImplementedError("write your pallas kernel here")


# speedup vs baseline: 20.4929x; 20.4929x over previous
import functools

import jax
import jax.numpy as jnp
from jax import lax
from jax.experimental import pallas as pl
from jax.experimental.pallas import tpu as pltpu
from jax.experimental.pallas import tpu_sc as plsc

N = 10000
NG = 64
IN_DIM = 128
TCN = 32
GH = 48
HEADS = 2
E = 320000

NC = 2
NS = 16
NW = NC * NS
CHUNK = 128
NCHUNK = -(-E // (NW * CHUNK))
EW = NCHUNK * CHUNK
EP = EW * NW
NACC = -(-(N + 1) // 128) * 128
RPT = NACC // NS


def _axis(name):
    return lax.axis_index(name)


def _lrelu(v):
    return jnp.maximum(v, 0.2 * v)


def _elu(v):
    return jnp.where(v > 0, v, jnp.exp(jnp.minimum(v, 0.0)) - 1.0)


def _encoder_body(x_ref, b_ref, ea_ref, c1wt_ref, c1b_ref, c2wt_ref, c2b_ref,
                  bng_ref, bnb_ref, wl_ref, bl_ref, wr_ref, br_ref,
                  xl_ref, xr_ref, eam_ref):
    f32 = jnp.float32
    x = x_ref[...]
    b = b_ref[...]
    neg1 = jnp.full((1, 1), -1, jnp.int32)
    neg2 = jnp.full((1, 1), -2, jnp.int32)
    bp1 = jnp.concatenate([neg1, b[:-1]], axis=0)
    bn1 = jnp.concatenate([b[1:], neg2], axis=0)
    bp2 = jnp.concatenate([neg1, neg1, b[:-2]], axis=0)
    bn2 = jnp.concatenate([b[2:], neg2, neg2], axis=0)
    mp1 = (bp1 == b).astype(f32)
    mn1 = (bn1 == b).astype(f32)
    mp2 = (bp2 == b).astype(f32)
    mn2 = (bn2 == b).astype(f32)

    zrow = jnp.zeros((1, x.shape[1]), f32)
    xp = jnp.concatenate([zrow, x[:-1]], axis=0) * mp1
    xn = jnp.concatenate([x[1:], zrow], axis=0) * mn1
    dot = functools.partial(jnp.dot, preferred_element_type=f32)
    h = dot(xp, c1wt_ref[0]) + dot(x, c1wt_ref[1]) + dot(xn, c1wt_ref[2])
    h = jnp.maximum(h + c1b_ref[...], 0.0)

    zrow2 = jnp.zeros((2, h.shape[1]), f32)
    hp = jnp.concatenate([zrow2, h[:-2]], axis=0) * mp2
    hn = jnp.concatenate([h[2:], zrow2], axis=0) * mn2
    h2 = dot(hp, c2wt_ref[0]) + dot(h, c2wt_ref[1]) + dot(hn, c2wt_ref[2])
    h2 = jnp.maximum(h2 + c2b_ref[...], 0.0)

    gids = lax.broadcasted_iota(jnp.int32, (1, NG), 1)
    onehot = (b == gids).astype(f32)
    counts = jnp.sum(onehot, axis=0, keepdims=True)
    cnt = jnp.maximum(counts, 1.0).reshape(NG, 1)
    dnum = (((0,), (0,)), ((), ()))
    sums = lax.dot_general(onehot, h2, dnum, preferred_element_type=f32)
    sumsq = lax.dot_general(onehot, h2 * h2, dnum, preferred_element_type=f32)
    mean = sums / cnt
    var = sumsq / cnt - mean * mean
    scale = bng_ref[...] / jnp.sqrt(var + 1e-5)
    shift = bnb_ref[...] - mean * scale
    hq = h2 * dot(onehot, scale) + dot(onehot, shift)

    xl_ref[...] = dot(hq, wl_ref[...]) + bl_ref[...]
    xr_ref[...] = dot(hq, wr_ref[...]) + br_ref[...]
    colsum = jnp.sum(ea_ref[...], axis=0, keepdims=True)
    col = lax.broadcasted_iota(jnp.int32, (1, 128), 1)
    even = (col % 2) == 0
    s0 = jnp.sum(jnp.where(even, colsum, 0.0), axis=1, keepdims=True)
    s1 = jnp.sum(jnp.where(even, 0.0, colsum), axis=1, keepdims=True)
    eam_ref[...] = jnp.concatenate([s0, s1], axis=1) * (1.0 / E)


def _make_edge_pass(ch, heads, accw):
    nk = ch // 16
    hk = nk // heads
    mesh = plsc.VectorSubcoreMesh(core_axis_name="c", subcore_axis_name="s",
                                  num_cores=NC, num_subcores=NS)

    @functools.partial(
        pl.kernel,
        out_type=jax.ShapeDtypeStruct((NC, NACC, accw), jnp.float32),
        mesh=mesh,
        compiler_params=pltpu.CompilerParams(needs_layout_passes=False,
                                             use_tc_tiling_on_sc=False),
        scratch_types=[
            pltpu.VMEM_SHARED((NACC, accw), jnp.float32),
            pltpu.VMEM((CHUNK,), jnp.int32),
            pltpu.VMEM((CHUNK,), jnp.int32),
            pltpu.VMEM((CHUNK, 16), jnp.float32),
            pltpu.VMEM((CHUNK, 16), jnp.float32),
            pltpu.VMEM((CHUNK, ch), jnp.float32),
            pltpu.VMEM((CHUNK, ch), jnp.float32),
            pltpu.VMEM((CHUNK, accw), jnp.float32),
            pltpu.VMEM((2, ch), jnp.float32),
            pltpu.VMEM((ch,), jnp.float32),
        ],
    )
    def edge_pass(xl_hbm, xr_hbm, src_hbm, dst_hbm, ea0_hbm, ea1_hbm, we_hbm,
                  att_hbm, zr_hbm, out_hbm,
                  acc, sidx, didx, ea0, ea1, xlr, xrr, wbuf, wev, attv):
        cid = _axis("c")
        sid = _axis("s")
        wid = sid * NC + cid
        pltpu.sync_copy(we_hbm, wev)
        pltpu.sync_copy(att_hbm, attv)
        pltpu.sync_copy(zr_hbm.at[pl.ds(sid * RPT, RPT)],
                        acc.at[pl.ds(sid * RPT, RPT)])
        plsc.subcore_barrier()

        wv = [[wev[h, pl.ds(k * 16, 16)] for k in range(nk)] for h in (0, 1)]
        av = [attv[pl.ds(k * 16, 16)] for k in range(nk)]
        lane = lax.broadcasted_iota(jnp.int32, (16,), 0)

        def chunk_body(i, carry):
            base = wid * EW + i * CHUNK
            pltpu.sync_copy(src_hbm.at[pl.ds(base, CHUNK)], sidx)
            pltpu.sync_copy(dst_hbm.at[pl.ds(base, CHUNK)], didx)
            pltpu.sync_copy(ea0_hbm.at[pl.ds(base, CHUNK)], ea0)
            pltpu.sync_copy(ea1_hbm.at[pl.ds(base, CHUNK)], ea1)
            pltpu.sync_copy(xl_hbm.at[sidx], xlr)
            pltpu.sync_copy(xr_hbm.at[didx], xrr)

            def edge_body(e, c2):
                a0 = ea0[e]
                a1 = ea1[e]
                xls = [xlr[e, pl.ds(k * 16, 16)] for k in range(nk)]
                xrs = [xrr[e, pl.ds(k * 16, 16)] for k in range(nk)]
                exs = []
                for h in range(heads):
                    tacc = None
                    for j in range(hk):
                        k = h * hk + j
                        m = xls[k] + xrs[k] + a0 * wv[0][k] + a1 * wv[1][k]
                        t = _lrelu(m) * av[k]
                        tacc = t if tacc is None else tacc + t
                    s = jnp.sum(tacc)
                    ev = jnp.exp(jnp.broadcast_to(s, (16,)))
                    exs.append(ev)
                    for j in range(hk):
                        k = h * hk + j
                        wbuf[e, pl.ds(k * 16, 16)] = xls[k] * ev
                exv = jnp.where(lane == 0, exs[0], 0.0)
                for h in range(1, heads):
                    exv = jnp.where(lane == h, exs[h], exv)
                wbuf[e, pl.ds(ch, 16)] = exv
                return c2

            lax.fori_loop(0, CHUNK, edge_body, 0)
            pltpu.sync_copy(wbuf, acc.at[didx], add=True)
            return carry

        lax.fori_loop(0, NCHUNK, chunk_body, 0)
        plsc.subcore_barrier()
        pltpu.sync_copy(acc.at[pl.ds(sid * RPT, RPT)],
                        out_hbm.at[cid, pl.ds(sid * RPT, RPT)])

    return edge_pass


def _mid_body(acc_ref, xl1_ref, xr1_ref, eam_ref, we1_ref, att1_ref, b1_ref,
              wl2_ref, bl2_ref, wr2_ref, br2_ref, we2_ref, att2_ref,
              xl2_ref, xr2_ref, sn2_ref, sd2_ref):
    f32 = jnp.float32
    dot = functools.partial(jnp.dot, preferred_element_type=f32)
    num = acc_ref[0, :N, :HEADS * GH] + acc_ref[1, :N, :HEADS * GH]
    den = (acc_ref[0, :N, HEADS * GH:HEADS * GH + HEADS]
           + acc_ref[1, :N, HEADS * GH:HEADS * GH + HEADS])
    xl1 = xl1_ref[...]
    xr1 = xr1_ref[...]
    eam = eam_ref[...]
    efs = jnp.sum(eam * we1_ref[...], axis=0, keepdims=True)
    ts = _lrelu(xl1 + xr1 + efs) * att1_ref[...]
    e0 = jnp.exp(jnp.sum(ts[:, :GH], axis=1, keepdims=True))
    e1 = jnp.exp(jnp.sum(ts[:, GH:], axis=1, keepdims=True))
    numt = num + jnp.concatenate([xl1[:, :GH] * e0, xl1[:, GH:] * e1], axis=1)
    d0 = den[:, 0:1] + e0 + 1e-16
    d1 = den[:, 1:2] + e1 + 1e-16
    o = jnp.concatenate([numt[:, :GH] / d0, numt[:, GH:] / d1], axis=1)
    h1 = _elu(o + b1_ref[...])

    xl2 = dot(h1, wl2_ref[...]) + bl2_ref[...]
    xr2 = dot(h1, wr2_ref[...]) + br2_ref[...]
    efs2 = jnp.sum(eam * we2_ref[...], axis=0, keepdims=True)
    ts2 = _lrelu(xl2 + xr2 + efs2) * att2_ref[...]
    e2 = jnp.exp(jnp.sum(ts2, axis=1, keepdims=True))
    xl2_ref[...] = xl2
    xr2_ref[...] = xr2
    sn2_ref[...] = xl2 * e2
    sd2_ref[...] = e2


def _final_body(acc_ref, sn2_ref, sd2_ref, b2_ref, batch_ref,
                fc1w_ref, fc1b_ref, fc2w_ref, fc2b_ref, out_ref, mp_ref):
    f32 = jnp.float32
    dot = functools.partial(jnp.dot, preferred_element_type=f32)
    num = acc_ref[0, :N, :GH] + acc_ref[1, :N, :GH] + sn2_ref[...]
    den = acc_ref[0, :N, GH:GH + 1] + acc_ref[1, :N, GH:GH + 1] + sd2_ref[...]
    h = _elu(num / (den + 1e-16) + b2_ref[...])

    b = batch_ref[...]
    gids = lax.broadcasted_iota(jnp.int32, (1, NG), 1)
    onehot = (b == gids).astype(f32)
    counts = jnp.sum(onehot, axis=0, keepdims=True)
    cnt = jnp.maximum(counts, 1.0).reshape(NG, 1)
    dnum = (((0,), (0,)), ((), ()))
    mean_pool = lax.dot_general(onehot, h, dnum,
                                preferred_element_type=f32) / cnt
    neg = jnp.float32(-jnp.inf)

    def mp_body(g, carry):
        mask = b == g
        mp_ref[pl.ds(g, 1), :] = jnp.max(jnp.where(mask, h, neg), axis=0,
                                         keepdims=True)
        return carry

    lax.fori_loop(0, NG, mp_body, 0)
    max_pool = mp_ref[...]
    pooled = jnp.concatenate([mean_pool, max_pool], axis=1)
    hid = jnp.maximum(dot(pooled, fc1w_ref[...]) + fc1b_ref[...], 0.0)
    out_ref[...] = dot(hid, fc2w_ref[...]) + fc2b_ref[...]


def kernel(x, edge_index, edge_attr, batch, c1w, c1b, c2w, c2b, bn_g, bn_b,
           g1_wl, g1_bl, g1_wr, g1_br, g1_we, g1_att, g1_bias,
           g2_wl, g2_bl, g2_wr, g2_br, g2_we, g2_att, g2_bias,
           fc1_w, fc1_b, fc2_w, fc2_b):
    f32 = jnp.float32
    batch2 = batch.reshape(N, 1)
    c1wt = jnp.transpose(c1w, (2, 1, 0))
    c2wt = jnp.transpose(c2w, (2, 1, 0))

    sds = jax.ShapeDtypeStruct
    tc_params = pltpu.CompilerParams(vmem_limit_bytes=100 * 1024 * 1024)
    xl1, xr1, eam = pl.pallas_call(
        _encoder_body,
        out_shape=[sds((N, HEADS * GH), f32), sds((N, HEADS * GH), f32),
                   sds((1, 2), f32)],
        compiler_params=tc_params,
    )(x, batch2, edge_attr.reshape(E * 2 // 128, 128), c1wt,
      c1b.reshape(1, TCN), c2wt,
      c2b.reshape(1, TCN), bn_g.reshape(1, TCN), bn_b.reshape(1, TCN),
      g1_wl, g1_bl.reshape(1, HEADS * GH), g1_wr, g1_br.reshape(1, HEADS * GH))

    pad = EP - E
    src_p = jnp.concatenate([edge_index[0], jnp.zeros((pad,), jnp.int32)])
    dst_p = jnp.concatenate([edge_index[1],
                             jnp.full((pad,), N, jnp.int32)])
    zpad = jnp.zeros((pad,), f32)
    ea0_p = jnp.broadcast_to(
        jnp.concatenate([edge_attr[:, 0], zpad])[:, None], (EP, 16))
    ea1_p = jnp.broadcast_to(
        jnp.concatenate([edge_attr[:, 1], zpad])[:, None], (EP, 16))
    zr1 = jnp.zeros((NACC, HEADS * GH + 16), f32)
    zr2 = jnp.zeros((NACC, GH + 16), f32)
    padrows = jnp.zeros((NACC - N, HEADS * GH), f32)

    edge1 = _make_edge_pass(HEADS * GH, HEADS, HEADS * GH + 16)
    acc1 = edge1(jnp.concatenate([xl1, padrows], axis=0),
                 jnp.concatenate([xr1, padrows], axis=0),
                 src_p, dst_p, ea0_p, ea1_p, g1_we,
                 g1_att.reshape(HEADS * GH), zr1)

    xl2, xr2, sn2, sd2 = pl.pallas_call(
        _mid_body,
        out_shape=[sds((N, GH), f32), sds((N, GH), f32), sds((N, GH), f32),
                   sds((N, 1), f32)],
        compiler_params=tc_params,
    )(acc1, xl1, xr1, eam.reshape(2, 1), g1_we,
      g1_att.reshape(1, HEADS * GH), g1_bias.reshape(1, HEADS * GH),
      g2_wl, g2_bl.reshape(1, GH), g2_wr, g2_br.reshape(1, GH), g2_we,
      g2_att.reshape(1, GH))

    padrows2 = jnp.zeros((NACC - N, GH), f32)
    edge2 = _make_edge_pass(GH, 1, GH + 16)
    acc2 = edge2(jnp.concatenate([xl2, padrows2], axis=0),
                 jnp.concatenate([xr2, padrows2], axis=0),
                 src_p, dst_p, ea0_p, ea1_p, g2_we, g2_att.reshape(GH), zr2)

    out = pl.pallas_call(
        _final_body,
        out_shape=sds((NG, 1), f32),
        compiler_params=tc_params,
        scratch_shapes=[pltpu.VMEM((NG, GH), f32)],
    )(acc2, sn2, sd2, g2_bias.reshape(1, GH), batch2,
      fc1_w, fc1_b.reshape(1, 128), fc2_w, fc2_b.reshape(1, 1))
    return out[:, 0]


# speedup vs baseline: 22.3580x; 1.0910x over previous
import functools

import jax
import jax.numpy as jnp
from jax import lax
from jax.experimental import pallas as pl
from jax.experimental.pallas import tpu as pltpu
from jax.experimental.pallas import tpu_sc as plsc

N = 10000
NG = 64
IN_DIM = 128
TCN = 32
GH = 48
HEADS = 2
E = 320000

NC = 2
NS = 16
NW = NC * NS
CHUNK = 128
NCHUNK = -(-E // (NW * CHUNK))
NCHUNK += NCHUNK % 2
EW = NCHUNK * CHUNK
EP = EW * NW
NACC = -(-(N + 1) // 128) * 128
RPT = NACC // NS


def _axis(name):
    return lax.axis_index(name)


def _lrelu(v):
    return jnp.maximum(v, 0.2 * v)


def _elu(v):
    return jnp.where(v > 0, v, jnp.exp(jnp.minimum(v, 0.0)) - 1.0)


def _encoder_body(x_ref, b_ref, ea_ref, c1wt_ref, c1b_ref, c2wt_ref, c2b_ref,
                  bng_ref, bnb_ref, wl_ref, bl_ref, wr_ref, br_ref,
                  xl_ref, xr_ref, eam_ref, sums_ref, sumsq_ref):
    f32 = jnp.float32
    x = x_ref[...]
    b = b_ref[...]
    neg1 = jnp.full((1, 1), -1, jnp.int32)
    neg2 = jnp.full((1, 1), -2, jnp.int32)
    bp1 = jnp.concatenate([neg1, b[:-1]], axis=0)
    bn1 = jnp.concatenate([b[1:], neg2], axis=0)
    bp2 = jnp.concatenate([neg1, neg1, b[:-2]], axis=0)
    bn2 = jnp.concatenate([b[2:], neg2, neg2], axis=0)
    mp1 = (bp1 == b).astype(f32)
    mn1 = (bn1 == b).astype(f32)
    mp2 = (bp2 == b).astype(f32)
    mn2 = (bn2 == b).astype(f32)

    zrow = jnp.zeros((1, x.shape[1]), f32)
    xp = jnp.concatenate([zrow, x[:-1]], axis=0) * mp1
    xn = jnp.concatenate([x[1:], zrow], axis=0) * mn1
    dot = functools.partial(jnp.dot, preferred_element_type=f32)
    h = dot(jnp.concatenate([xp, x, xn], axis=1), c1wt_ref[...])
    h = jnp.maximum(h + c1b_ref[...], 0.0)

    zrow2 = jnp.zeros((2, h.shape[1]), f32)
    hp = jnp.concatenate([zrow2, h[:-2]], axis=0) * mp2
    hn = jnp.concatenate([h[2:], zrow2], axis=0) * mn2
    h2 = dot(jnp.concatenate([hp, h, hn], axis=1), c2wt_ref[...])
    h2 = jnp.maximum(h2 + c2b_ref[...], 0.0)

    gids = lax.broadcasted_iota(jnp.int32, (1, NG), 1)
    onehot = (b == gids).astype(f32)
    counts = jnp.sum(onehot, axis=0, keepdims=True)
    cnt = jnp.maximum(counts, 1.0).reshape(NG, 1)

    def sum_body(g, carry):
        mask = b == g
        sums_ref[pl.ds(g, 1), :] = jnp.sum(jnp.where(mask, h2, 0.0), axis=0,
                                           keepdims=True)
        return carry

    lax.fori_loop(0, NG, sum_body, 0)
    mean = sums_ref[...] / cnt
    sums_ref[...] = mean

    def gather_mean(g, acc):
        return jnp.where(b == g, sums_ref[pl.ds(g, 1), :], acc)

    mean_n = lax.fori_loop(0, NG, gather_mean, jnp.zeros((N, TCN), f32))
    hc = h2 - mean_n
    hc2 = hc * hc

    def sumsq_body(g, carry):
        mask = b == g
        sumsq_ref[pl.ds(g, 1), :] = jnp.sum(jnp.where(mask, hc2, 0.0), axis=0,
                                            keepdims=True)
        return carry

    lax.fori_loop(0, NG, sumsq_body, 0)
    var = sumsq_ref[...] / cnt
    a = var + 1e-5
    r = lax.rsqrt(a)
    r = r * (1.5 - 0.5 * a * r * r)
    r = r * (1.5 - 0.5 * a * r * r)
    scale = bng_ref[...] * r
    sumsq_ref[...] = scale

    def gather_scale(g, acc):
        return jnp.where(b == g, sumsq_ref[pl.ds(g, 1), :], acc)

    scale_n = lax.fori_loop(0, NG, gather_scale, jnp.zeros((N, TCN), f32))
    hq = hc * scale_n + bnb_ref[...]

    xl_ref[...] = dot(hq, wl_ref[...]) + bl_ref[...]
    xr_ref[...] = dot(hq, wr_ref[...]) + br_ref[...]
    colsum = jnp.sum(ea_ref[...], axis=0, keepdims=True)
    col = lax.broadcasted_iota(jnp.int32, (1, 128), 1)
    even = (col % 2) == 0
    s0 = jnp.sum(jnp.where(even, colsum, 0.0), axis=1, keepdims=True)
    s1 = jnp.sum(jnp.where(even, 0.0, colsum), axis=1, keepdims=True)
    eam_ref[...] = jnp.concatenate([s0, s1], axis=1) * (1.0 / E)


def _make_edge_pass(ch, heads, accw, ck):
    nk = ch // 16
    hk = nk // heads
    nchunk = EW // ck
    mesh = plsc.VectorSubcoreMesh(core_axis_name="c", subcore_axis_name="s",
                                  num_cores=NC, num_subcores=NS)

    @functools.partial(
        pl.kernel,
        out_type=jax.ShapeDtypeStruct((NC, NACC, accw), jnp.float32),
        mesh=mesh,
        compiler_params=pltpu.CompilerParams(needs_layout_passes=False,
                                             use_tc_tiling_on_sc=False),
        scratch_types=(
            [pltpu.VMEM_SHARED((NACC, accw), jnp.float32)]
            + [pltpu.VMEM((ck,), jnp.int32)] * 6
            + [pltpu.VMEM((ck, 16), jnp.float32)] * 4
            + [pltpu.VMEM((ck, ch), jnp.float32)] * 4
            + [pltpu.VMEM((ck, accw), jnp.float32)] * 2
            + [pltpu.VMEM((2, ch), jnp.float32),
               pltpu.VMEM((ch,), jnp.float32)]
            + [pltpu.SemaphoreType.DMA] * 6
        ),
    )
    def edge_pass(xl_hbm, xr_hbm, src_hbm, dst_hbm, ea0_hbm, ea1_hbm, we_hbm,
                  att_hbm, zr_hbm, out_hbm,
                  acc, sidx0, sidx1, didx0, didx1, dsc0, dsc1,
                  ea00, ea01, ea10, ea11, xlr0, xlr1, xrr0, xrr1,
                  wbuf0, wbuf1, wev, attv,
                  smi0, smi1, smg0, smg1, sms0, sms1):
        cid = _axis("c")
        sid = _axis("s")
        wid = sid * NC + cid
        pltpu.sync_copy(we_hbm, wev)
        pltpu.sync_copy(att_hbm, attv)
        pltpu.sync_copy(zr_hbm.at[pl.ds(sid * RPT, RPT)],
                        acc.at[pl.ds(sid * RPT, RPT)])
        plsc.subcore_barrier()

        wv = [[wev[h, pl.ds(k * 16, 16)] for k in range(nk)] for h in (0, 1)]
        av = [attv[pl.ds(k * 16, 16)] for k in range(nk)]
        lane = lax.broadcasted_iota(jnp.int32, (16,), 0)

        slots = [
            (sidx0, didx0, dsc0, ea00, ea10, xlr0, xrr0, wbuf0,
             smi0, smg0, sms0),
            (sidx1, didx1, dsc1, ea01, ea11, xlr1, xrr1, wbuf1,
             smi1, smg1, sms1),
        ]

        def issue_idx(s, base):
            sidx, didx, _, ea0, ea1, _, _, _, smi, _, _ = slots[s]
            pltpu.async_copy(src_hbm.at[pl.ds(base, ck)], sidx, smi)
            pltpu.async_copy(dst_hbm.at[pl.ds(base, ck)], didx, smi)
            pltpu.async_copy(ea0_hbm.at[pl.ds(base, ck)], ea0, smi)
            pltpu.async_copy(ea1_hbm.at[pl.ds(base, ck)], ea1, smi)

        def wait_idx(s, base):
            sidx, didx, _, ea0, ea1, _, _, _, smi, _, _ = slots[s]
            pltpu.make_async_copy(src_hbm.at[pl.ds(base, ck)], sidx,
                                  smi).wait()
            pltpu.make_async_copy(dst_hbm.at[pl.ds(base, ck)], didx,
                                  smi).wait()
            pltpu.make_async_copy(ea0_hbm.at[pl.ds(base, ck)], ea0,
                                  smi).wait()
            pltpu.make_async_copy(ea1_hbm.at[pl.ds(base, ck)], ea1,
                                  smi).wait()

        def issue_gather(s):
            sidx, didx, _, _, _, xlr, xrr, _, _, smg, _ = slots[s]
            pltpu.async_copy(xl_hbm.at[sidx], xlr, smg)
            pltpu.async_copy(xr_hbm.at[didx], xrr, smg)

        def wait_gather(s):
            sidx, didx, _, _, _, xlr, xrr, _, _, smg, _ = slots[s]
            pltpu.make_async_copy(xl_hbm.at[sidx], xlr, smg).wait()
            pltpu.make_async_copy(xr_hbm.at[didx], xrr, smg).wait()

        def issue_scatter(s):
            _, _, dsc, _, _, _, _, wbuf, _, _, sms = slots[s]
            pltpu.async_copy(wbuf, acc.at[dsc], sms, add=True)

        def wait_scatter(s):
            _, _, dsc, _, _, _, _, wbuf, _, _, sms = slots[s]
            pltpu.make_async_copy(wbuf, acc.at[dsc], sms).wait()

        def compute(s):
            _, didx, dsc, ea0, ea1, xlr, xrr, wbuf, _, _, _ = slots[s]
            for g in range(ck // 16):
                dsc[pl.ds(g * 16, 16)] = didx[pl.ds(g * 16, 16)]

            def edge_body(e, c2):
                a0 = ea0[e]
                a1 = ea1[e]
                xls = [xlr[e, pl.ds(k * 16, 16)] for k in range(nk)]
                xrs = [xrr[e, pl.ds(k * 16, 16)] for k in range(nk)]
                exs = []
                for h in range(heads):
                    tacc = None
                    for j in range(hk):
                        k = h * hk + j
                        m = xls[k] + xrs[k] + a0 * wv[0][k] + a1 * wv[1][k]
                        t = _lrelu(m) * av[k]
                        tacc = t if tacc is None else tacc + t
                    sv = jnp.sum(tacc)
                    ev = jnp.exp(jnp.broadcast_to(sv, (16,)))
                    exs.append(ev)
                    for j in range(hk):
                        k = h * hk + j
                        wbuf[e, pl.ds(k * 16, 16)] = xls[k] * ev
                exv = jnp.where(lane == 0, exs[0], 0.0)
                for h in range(1, heads):
                    exv = jnp.where(lane == h, exs[h], exv)
                wbuf[e, pl.ds(ch, 16)] = exv
                return c2

            lax.fori_loop(0, ck, edge_body, 0)

        base00 = wid * EW
        issue_idx(0, base00)
        issue_idx(1, base00 + ck)
        wait_idx(0, base00)
        issue_gather(0)

        def pipe_body(t, carry):
            i0 = 2 * t
            base0 = wid * EW + i0 * ck
            base1 = base0 + ck

            @pl.when(t >= 1)
            def _():
                wait_scatter(0)
            wait_gather(0)
            wait_idx(1, base1)
            issue_gather(1)
            compute(0)
            issue_scatter(0)

            @pl.when(i0 + 2 < nchunk)
            def _():
                issue_idx(0, base0 + 2 * ck)

            @pl.when(t >= 1)
            def _():
                wait_scatter(1)
            wait_gather(1)

            @pl.when(i0 + 2 < nchunk)
            def _():
                wait_idx(0, base0 + 2 * ck)
                issue_gather(0)
            compute(1)
            issue_scatter(1)

            @pl.when(i0 + 3 < nchunk)
            def _():
                issue_idx(1, base1 + 2 * ck)
            return carry

        lax.fori_loop(0, nchunk // 2, pipe_body, 0)
        wait_scatter(0)
        wait_scatter(1)
        plsc.subcore_barrier()
        pltpu.sync_copy(acc.at[pl.ds(sid * RPT, RPT)],
                        out_hbm.at[cid, pl.ds(sid * RPT, RPT)])

    return edge_pass


def _mid_body(acc_ref, xl1_ref, xr1_ref, eam_ref, we1_ref, att1_ref, b1_ref,
              wl2_ref, bl2_ref, wr2_ref, br2_ref, we2_ref, att2_ref,
              xl2_ref, xr2_ref, sn2_ref, sd2_ref):
    f32 = jnp.float32
    dot = functools.partial(jnp.dot, preferred_element_type=f32)
    num = acc_ref[0, :N, :HEADS * GH] + acc_ref[1, :N, :HEADS * GH]
    den = (acc_ref[0, :N, HEADS * GH:HEADS * GH + HEADS]
           + acc_ref[1, :N, HEADS * GH:HEADS * GH + HEADS])
    xl1 = xl1_ref[...]
    xr1 = xr1_ref[...]
    eam = eam_ref[...]
    efs = jnp.sum(eam * we1_ref[...], axis=0, keepdims=True)
    ts = _lrelu(xl1 + xr1 + efs) * att1_ref[...]
    e0 = jnp.exp(jnp.sum(ts[:, :GH], axis=1, keepdims=True))
    e1 = jnp.exp(jnp.sum(ts[:, GH:], axis=1, keepdims=True))
    numt = num + jnp.concatenate([xl1[:, :GH] * e0, xl1[:, GH:] * e1], axis=1)
    d0 = den[:, 0:1] + e0 + 1e-16
    d1 = den[:, 1:2] + e1 + 1e-16
    o = jnp.concatenate([numt[:, :GH] / d0, numt[:, GH:] / d1], axis=1)
    h1 = _elu(o + b1_ref[...])

    xl2 = dot(h1, wl2_ref[...]) + bl2_ref[...]
    xr2 = dot(h1, wr2_ref[...]) + br2_ref[...]
    efs2 = jnp.sum(eam * we2_ref[...], axis=0, keepdims=True)
    ts2 = _lrelu(xl2 + xr2 + efs2) * att2_ref[...]
    e2 = jnp.exp(jnp.sum(ts2, axis=1, keepdims=True))
    xl2_ref[...] = xl2
    xr2_ref[...] = xr2
    sn2_ref[...] = xl2 * e2
    sd2_ref[...] = e2


def _final_body(acc_ref, sn2_ref, sd2_ref, b2_ref, batch_ref,
                fc1w_ref, fc1b_ref, fc2w_ref, fc2b_ref, out_ref, mp_ref):
    f32 = jnp.float32
    dot = functools.partial(jnp.dot, preferred_element_type=f32)
    num = acc_ref[0, :N, :GH] + acc_ref[1, :N, :GH] + sn2_ref[...]
    den = acc_ref[0, :N, GH:GH + 1] + acc_ref[1, :N, GH:GH + 1] + sd2_ref[...]
    h = _elu(num / (den + 1e-16) + b2_ref[...])

    b = batch_ref[...]
    gids = lax.broadcasted_iota(jnp.int32, (1, NG), 1)
    onehot = (b == gids).astype(f32)
    counts = jnp.sum(onehot, axis=0, keepdims=True)
    cnt = jnp.maximum(counts, 1.0).reshape(NG, 1)
    dnum = (((0,), (0,)), ((), ()))
    mean_pool = lax.dot_general(onehot, h, dnum,
                                preferred_element_type=f32) / cnt
    neg = jnp.float32(-jnp.inf)

    def mp_body(g, carry):
        mask = b == g
        mp_ref[pl.ds(g, 1), :] = jnp.max(jnp.where(mask, h, neg), axis=0,
                                         keepdims=True)
        return carry

    lax.fori_loop(0, NG, mp_body, 0)
    max_pool = mp_ref[...]
    pooled = jnp.concatenate([mean_pool, max_pool], axis=1)
    hid = jnp.maximum(dot(pooled, fc1w_ref[...]) + fc1b_ref[...], 0.0)
    out_ref[...] = dot(hid, fc2w_ref[...]) + fc2b_ref[...]


def kernel(x, edge_index, edge_attr, batch, c1w, c1b, c2w, c2b, bn_g, bn_b,
           g1_wl, g1_bl, g1_wr, g1_br, g1_we, g1_att, g1_bias,
           g2_wl, g2_bl, g2_wr, g2_br, g2_we, g2_att, g2_bias,
           fc1_w, fc1_b, fc2_w, fc2_b):
    f32 = jnp.float32
    batch2 = batch.reshape(N, 1)
    c1wt = jnp.transpose(c1w, (2, 1, 0)).reshape(3 * IN_DIM, TCN)
    c2wt = jnp.transpose(c2w, (2, 1, 0)).reshape(3 * TCN, TCN)

    sds = jax.ShapeDtypeStruct
    tc_params = pltpu.CompilerParams(vmem_limit_bytes=100 * 1024 * 1024)
    xl1, xr1, eam = pl.pallas_call(
        _encoder_body,
        out_shape=[sds((N, HEADS * GH), f32), sds((N, HEADS * GH), f32),
                   sds((1, 2), f32)],
        compiler_params=tc_params,
        scratch_shapes=[pltpu.VMEM((NG, TCN), f32)] * 2,
    )(x, batch2, edge_attr.reshape(E * 2 // 128, 128), c1wt,
      c1b.reshape(1, TCN), c2wt,
      c2b.reshape(1, TCN), bn_g.reshape(1, TCN), bn_b.reshape(1, TCN),
      g1_wl, g1_bl.reshape(1, HEADS * GH), g1_wr, g1_br.reshape(1, HEADS * GH))

    pad = EP - E
    src_p = jnp.concatenate([edge_index[0], jnp.zeros((pad,), jnp.int32)])
    dst_p = jnp.concatenate([edge_index[1],
                             jnp.full((pad,), N, jnp.int32)])
    zpad = jnp.zeros((pad,), f32)
    ea0_p = jnp.broadcast_to(
        jnp.concatenate([edge_attr[:, 0], zpad])[:, None], (EP, 16))
    ea1_p = jnp.broadcast_to(
        jnp.concatenate([edge_attr[:, 1], zpad])[:, None], (EP, 16))
    zr1 = jnp.zeros((NACC, HEADS * GH + 16), f32)
    zr2 = jnp.zeros((NACC, GH + 16), f32)
    padrows = jnp.zeros((NACC - N, HEADS * GH), f32)

    edge1 = _make_edge_pass(HEADS * GH, HEADS, HEADS * GH + 16, 64)
    acc1 = edge1(jnp.concatenate([xl1, padrows], axis=0),
                 jnp.concatenate([xr1, padrows], axis=0),
                 src_p, dst_p, ea0_p, ea1_p, g1_we,
                 g1_att.reshape(HEADS * GH), zr1)

    xl2, xr2, sn2, sd2 = pl.pallas_call(
        _mid_body,
        out_shape=[sds((N, GH), f32), sds((N, GH), f32), sds((N, GH), f32),
                   sds((N, 1), f32)],
        compiler_params=tc_params,
    )(acc1, xl1, xr1, eam.reshape(2, 1), g1_we,
      g1_att.reshape(1, HEADS * GH), g1_bias.reshape(1, HEADS * GH),
      g2_wl, g2_bl.reshape(1, GH), g2_wr, g2_br.reshape(1, GH), g2_we,
      g2_att.reshape(1, GH))

    padrows2 = jnp.zeros((NACC - N, GH), f32)
    edge2 = _make_edge_pass(GH, 1, GH + 16, 128)
    acc2 = edge2(jnp.concatenate([xl2, padrows2], axis=0),
                 jnp.concatenate([xr2, padrows2], axis=0),
                 src_p, dst_p, ea0_p, ea1_p, g2_we, g2_att.reshape(GH), zr2)

    out = pl.pallas_call(
        _final_body,
        out_shape=sds((NG, 1), f32),
        compiler_params=tc_params,
        scratch_shapes=[pltpu.VMEM((NG, GH), f32)],
    )(acc2, sn2, sd2, g2_bias.reshape(1, GH), batch2,
      fc1_w, fc1_b.reshape(1, 128), fc2_w, fc2_b.reshape(1, 1))
    return out[:, 0]
